# baseline (speedup 1.0000x reference)
import functools

import jax
import jax.numpy as jnp
from jax import lax
from jax.experimental import pallas as pl
from jax.experimental.pallas import tpu as pltpu
from jax.experimental.pallas import tpu_sc as plsc

B = 16384
F = 26
E = 16
FIELD_DIM = 100000
TOTAL_ROWS = F * FIELD_DIM

NW = 32
BPW = B // NW
CB = 64
NCHUNK = BPW // CB
ROWS = CB * F
GPC = ROWS // 128
IDX_ROWS = BPW * F // 128
NPAIR = NCHUNK // 2


def _body(idx_hbm, embed_hbm, fc_hbm, bias_hbm, out_hbm,
          idx_v, rows0, rows1, fcv0, fcv1, t_v, out_v, bias_v, sem0, sem1):
    cid = lax.axis_index("c")
    sid = lax.axis_index("s")
    wid = sid * 2 + cid

    pltpu.sync_copy(idx_hbm.at[wid], idx_v)
    pltpu.sync_copy(bias_hbm, bias_v)
    bias_vec = bias_v[...]

    def fire(c, rows, fcv, sem):
        for j in range(GPC):
            r = c * GPC + j
            pltpu.async_copy(embed_hbm.at[idx_v.at[r]],
                             rows.at[pl.ds(j * 128, 128)], sem)
            pltpu.async_copy(fc_hbm.at[idx_v.at[r]],
                             fcv.at[pl.ds(j * 128, 128)], sem)

    def drain(rows, fcv, sem):
        pltpu.make_async_copy(embed_hbm.at[pl.ds(0, ROWS)], rows, sem).wait()
        pltpu.make_async_copy(fc_hbm.at[pl.ds(0, ROWS)], fcv, sem).wait()

    iot = lax.iota(jnp.int32, 16)

    def compute(c, rows, fcv):
        def item_body(i, carry):
            r0 = i * F
            s = rows[r0]
            ss = s * s
            for f in range(1, F):
                r = rows[r0 + f]
                s = s + r
                ss = ss + r * r
            t_v[pl.ds(i * E, E)] = s * s - ss
            return carry
        lax.fori_loop(0, CB, item_body, 0, unroll=2)

        for g in range(CB // 16):
            items = g * 16 + iot
            tidx = items * E
            tsum = plsc.load_gather(t_v, [tidx])
            for e in range(1, E):
                tsum = tsum + plsc.load_gather(t_v, [tidx + e])
            fidx = items * F
            fsum = plsc.load_gather(fcv, [fidx])
            for f in range(1, F):
                fsum = fsum + plsc.load_gather(fcv, [fidx + f])
            y = 0.5 * tsum + fsum + bias_vec
            sig = 1.0 / (1.0 + jnp.exp(-y))
            out_v[pl.ds(c * CB + g * 16, 16)] = sig

    fire(0, rows0, fcv0, sem0)

    def pair_body(cc, carry):
        c0 = 2 * cc
        fire(c0 + 1, rows1, fcv1, sem1)
        drain(rows0, fcv0, sem0)
        compute(c0, rows0, fcv0)

        @pl.when(cc < NPAIR - 1)
        def _():
            fire(c0 + 2, rows0, fcv0, sem0)

        drain(rows1, fcv1, sem1)
        compute(c0 + 1, rows1, fcv1)
        return carry
    lax.fori_loop(0, NPAIR, pair_body, 0)

    pltpu.sync_copy(out_v, out_hbm.at[pl.ds(wid * BPW, BPW)])


_sc_deepfm = functools.partial(
    pl.kernel,
    out_type=jax.ShapeDtypeStruct((B,), jnp.float32),
    mesh=plsc.VectorSubcoreMesh(core_axis_name="c", subcore_axis_name="s"),
    compiler_params=pltpu.CompilerParams(needs_layout_passes=False,
                                         use_tc_tiling_on_sc=False),
    scratch_types=[
        pltpu.VMEM((IDX_ROWS, 128), jnp.int32),
        pltpu.VMEM((ROWS, E), jnp.float32),
        pltpu.VMEM((ROWS, E), jnp.float32),
        pltpu.VMEM((ROWS,), jnp.float32),
        pltpu.VMEM((ROWS,), jnp.float32),
        pltpu.VMEM((CB * E,), jnp.float32),
        pltpu.VMEM((BPW,), jnp.float32),
        pltpu.VMEM((16,), jnp.float32),
        pltpu.SemaphoreType.DMA,
        pltpu.SemaphoreType.DMA,
    ],
)(_body)


@jax.jit
def kernel(xx, embed_table, fc_table, bias):
    offsets = (jnp.arange(F, dtype=jnp.int32) * FIELD_DIM)[None, :]
    idx = (xx.astype(jnp.int32) + offsets).reshape(NW, IDX_ROWS, 128)
    bias16 = jnp.broadcast_to(bias.astype(jnp.float32), (16,))
    fc_flat = fc_table.reshape(TOTAL_ROWS)
    return _sc_deepfm(idx, embed_table, fc_flat, bias16)
